# baseline (speedup 1.0000x reference)
import jax
import jax.numpy as jnp
from jax import lax
from jax.experimental import pallas as pl
from jax.experimental.pallas import tpu as pltpu
from jax.experimental.pallas import tpu_sc as plsc

_VOCAB = 1000000
_SEQ = 200
_DIM = 64
_BATCH = 4096
_SCALE = 8.0

_NC, _NS = 2, 16
_NW = _NC * _NS
_BW = _BATCH // _NW
_KT, _KR, _BT, _BL = _DIM // 8, 8, _BATCH // 128, 128
_NBUF = 4
_LOOK = 3
_LANES = 16


def _body(idxt_hbm, tt_hbm, post_hbm, out5_hbm,
          idxs_v, pos_v,
          g0, g1, g2, g3, o0, o1, o2, o3,
          gs0, gs1, gs2, gs3, os0, os1, os2, os3):
    wid = lax.axis_index("s") * _NC + lax.axis_index("c")
    G = (g0, g1, g2, g3)
    O = (o0, o1, o2, o3)
    GS = (gs0, gs1, gs2, gs3)
    OS = (os0, os1, os2, os3)

    pltpu.sync_copy(idxt_hbm.at[:, pl.ds(wid * _BW, _BW)], idxs_v)
    pltpu.sync_copy(post_hbm, pos_v)

    for s in range(_LOOK):
        pltpu.async_copy(tt_hbm.at[idxs_v.at[s]], G[s], GS[s])

    iota = jax.lax.iota(jnp.int32, _LANES)
    rows_c = [iota + _LANES * c for c in range(_BL // _LANES)]

    @pl.loop(0, _SEQ, step=_NBUF)
    def _outer(s0):
        for b in range(_NBUF):
            s = s0 + b

            @pl.when(s + _LOOK < _SEQ)
            def _fire():
                pltpu.async_copy(
                    tt_hbm.at[idxs_v.at[s + _LOOK]],
                    G[(b + _LOOK) % _NBUF], GS[(b + _LOOK) % _NBUF])

            pltpu.make_async_copy(
                tt_hbm.at[idxs_v.at[s]], G[b], GS[b]).wait()

            @pl.when(s >= _NBUF)
            def _wait_obuf():
                pltpu.make_async_copy(
                    O[b], out5_hbm.at[s - _NBUF, :, wid], OS[b]).wait()

            for kq in range(_DIM // _LANES):
                pos_chunk = pos_v[s, pl.ds(kq * _LANES, _LANES)]

                @plsc.parallel_loop(0, _LANES, unroll=8)
                def _k(kl):
                    k = kq * _LANES + kl
                    kt = kq * 2 + kl // _KR
                    kr = kl % _KR
                    pk = jnp.full((_LANES,), k, jnp.int32)
                    pv = pos_chunk.at[
                        jnp.full((_LANES,), kl, jnp.int32)
                    ].get(mode="promise_in_bounds")
                    for c in range(_BL // _LANES):
                        row = plsc.load_gather(G[b], [rows_c[c], pk])
                        O[b][kt, kr, pl.ds(c * _LANES, _LANES)] = (
                            row * _SCALE + pv)

            pltpu.async_copy(O[b], out5_hbm.at[s, :, wid], OS[b])

    for b in range(_NBUF):
        pltpu.make_async_copy(
            O[b], out5_hbm.at[_SEQ - _NBUF + b, :, wid], OS[b]).wait()


@jax.jit
def _embed5(idxt, token_table, pos_t):
    mesh = plsc.VectorSubcoreMesh(
        core_axis_name="c", subcore_axis_name="s",
        num_cores=_NC, num_subcores=_NS,
    )
    kern = pl.kernel(
        _body,
        out_type=jax.ShapeDtypeStruct((_SEQ, _KT, _BT, _KR, _BL), jnp.float32),
        mesh=mesh,
        compiler_params=pltpu.CompilerParams(
            use_tc_tiling_on_sc=False, needs_layout_passes=False),
        scratch_types=(
            [pltpu.VMEM((_SEQ, _BW), jnp.int32),
             pltpu.VMEM((_SEQ, _DIM), jnp.float32)]
            + [pltpu.VMEM((_BW, _DIM), jnp.float32) for _ in range(_NBUF)]
            + [pltpu.VMEM((_KT, _KR, _BL), jnp.float32) for _ in range(_NBUF)]
            + [pltpu.SemaphoreType.DMA for _ in range(2 * _NBUF)]
        ),
    )
    return kern(idxt, token_table, pos_t)


def kernel(inputs, token_table, pos_table):
    idxt = inputs.T.astype(jnp.int32)
    post = pos_table
    out5 = _embed5(idxt, token_table, post)
    out = jnp.transpose(out5, (2, 4, 0, 1, 3)).reshape(_BATCH, _SEQ, _DIM)
    return out
